# baseline (speedup 1.0000x reference)
import functools

import jax
import jax.numpy as jnp
from jax import lax
from jax.experimental import pallas as pl
from jax.experimental.pallas import tpu as pltpu
from jax.experimental.pallas import tpu_sc as plsc

B, T, D = 16, 2048, 64
N = B * T
NW = 32
PER_W = N // NW
CH = 256
NCH = PER_W // CH
OR_CH = CH // 2
LANES = 16


def _emb_body(idx_hbm, tok2_hbm, pos2_hbm, out2_hbm,
              idx_v, hidx_v, par_v, buf_v, pos_v, out_v, gsem, psem):
    cax = lax.axis_index("c")
    sax = lax.axis_index("s")
    wid = sax * 2 + cax

    for ch in range(NCH):
        jbase = pl.multiple_of(wid * PER_W + ch * CH, CH)
        rbase = pl.multiple_of(jbase // 2, OR_CH)
        prow = pl.multiple_of(rbase % (T // 2), OR_CH)

        pltpu.sync_copy(idx_hbm.at[pl.ds(jbase, CH)], idx_v)
        pos_cp = pltpu.async_copy(
            pos2_hbm.at[pl.ds(prow, OR_CH)], pos_v, psem)

        for q in range(CH // LANES):
            sl = pl.ds(q * LANES, LANES)
            v = idx_v[sl]
            hidx_v[sl] = lax.shift_right_logical(v, 1)
            par_v[sl] = (v & 1) * D

        g0 = pltpu.async_copy(
            tok2_hbm.at[hidx_v.at[pl.ds(0, 128)]],
            buf_v.at[pl.ds(0, 128), :], gsem)
        g1 = pltpu.async_copy(
            tok2_hbm.at[hidx_v.at[pl.ds(128, 128)]],
            buf_v.at[pl.ds(128, 128), :], gsem)
        g0.wait()
        g1.wait()
        pos_cp.wait()

        iota16 = lax.iota(jnp.int32, LANES)

        def row(r, _):
            j0 = 2 * r
            off0 = plsc.load_gather(
                par_v, [jnp.full((LANES,), j0, jnp.int32)])
            off1 = plsc.load_gather(
                par_v, [jnp.full((LANES,), j0 + 1, jnp.int32)])
            row0 = jnp.full((LANES,), j0, jnp.int32)
            row1 = jnp.full((LANES,), j0 + 1, jnp.int32)
            for q in range(D // LANES):
                sl = pl.ds(q * LANES, LANES)
                val = plsc.load_gather(
                    buf_v, [row0, off0 + q * LANES + iota16])
                out_v[r, sl] = val + pos_v[r, sl]
            for q in range(D // LANES):
                sl = pl.ds(D + q * LANES, LANES)
                val = plsc.load_gather(
                    buf_v, [row1, off1 + q * LANES + iota16])
                out_v[r, sl] = val + pos_v[r, sl]
            return _

        lax.fori_loop(0, OR_CH, row, 0)

        pltpu.sync_copy(out_v, out2_hbm.at[pl.ds(rbase, OR_CH)])


@jax.jit
def _emb(idx_flat, tok2, pos2):
    mesh = plsc.VectorSubcoreMesh(core_axis_name="c", subcore_axis_name="s")
    return pl.kernel(
        _emb_body,
        out_type=jax.ShapeDtypeStruct((N // 2, 2 * D), jnp.float32),
        mesh=mesh,
        scratch_types=[
            pltpu.VMEM((CH,), jnp.int32),
            pltpu.VMEM((CH,), jnp.int32),
            pltpu.VMEM((CH,), jnp.int32),
            pltpu.VMEM((CH, 2 * D), jnp.float32),
            pltpu.VMEM((OR_CH, 2 * D), jnp.float32),
            pltpu.VMEM((OR_CH, 2 * D), jnp.float32),
            pltpu.SemaphoreType.DMA,
            pltpu.SemaphoreType.DMA,
        ],
        compiler_params=pltpu.CompilerParams(needs_layout_passes=False),
    )(idx_flat, tok2, pos2)


def kernel(idx, tok_table, pos_table):
    tok2 = tok_table.reshape(tok_table.shape[0] // 2, 2 * D)
    pos2 = pos_table.reshape(T // 2, 2 * D)
    out2 = _emb(idx.reshape(N), tok2, pos2)
    return out2.reshape(B, T, D)
